# baseline (speedup 1.0000x reference)
import functools

import jax
import jax.numpy as jnp
from jax import lax
from jax.experimental import pallas as pl
from jax.experimental.pallas import tpu as pltpu
from jax.experimental.pallas import tpu_sc as plsc

N_CLS_CTX = 4
CTX_DIM = 512
CONTEXT_LEN = 77
PREFIX_LEN = 5
SUFFIX_LEN = CONTEXT_LEN - PREFIX_LEN - N_CLS_CTX
CTX_BEG = PREFIX_LEN
SUF_BEG = PREFIX_LEN + N_CLS_CTX


def _make_sc_gather(num_class: int, b: int, nc: int, b_per_w: int):
    @functools.partial(
        pl.kernel,
        mesh=plsc.VectorSubcoreMesh(core_axis_name="c", subcore_axis_name="s"),
        out_type=jax.ShapeDtypeStruct((b, N_CLS_CTX, CTX_DIM), jnp.float32),
        scratch_types=[
            pltpu.VMEM((b_per_w,), jnp.int32),
            pltpu.VMEM((b_per_w, N_CLS_CTX, CTX_DIM), jnp.float32),
            pltpu.SemaphoreType.DMA,
        ],
    )
    def gather(table_hbm, idx_hbm, out_hbm, idx_v, rows_v, sem):
        wid = lax.axis_index("s") * nc + lax.axis_index("c")
        base = wid * b_per_w
        pltpu.sync_copy(idx_hbm.at[pl.ds(base, b_per_w)], idx_v)
        pltpu.async_copy(table_hbm.at[idx_v], rows_v, sem).wait()
        pltpu.sync_copy(rows_v, out_hbm.at[pl.ds(base, b_per_w)])

    return gather


def _assemble_body(ctx_ref, pre_ref, suf_ref, out_ref):
    bb = out_ref.shape[1]
    out_ref[:PREFIX_LEN] = jnp.broadcast_to(
        pre_ref[...], (PREFIX_LEN, bb, CTX_DIM))
    out_ref[CTX_BEG:SUF_BEG] = ctx_ref[...]
    out_ref[SUF_BEG:] = jnp.broadcast_to(
        suf_ref[...], (SUFFIX_LEN, bb, CTX_DIM))


def _make_tc_assemble(b: int, bb: int):
    return pl.pallas_call(
        _assemble_body,
        grid=(b // bb,),
        in_specs=[
            pl.BlockSpec((N_CLS_CTX, bb, CTX_DIM), lambda i: (0, i, 0)),
            pl.BlockSpec((PREFIX_LEN, 1, CTX_DIM), lambda i: (0, 0, 0)),
            pl.BlockSpec((SUFFIX_LEN, 1, CTX_DIM), lambda i: (0, 0, 0)),
        ],
        out_specs=pl.BlockSpec((CONTEXT_LEN, bb, CTX_DIM), lambda i: (0, i, 0)),
        out_shape=jax.ShapeDtypeStruct((CONTEXT_LEN, b, CTX_DIM), jnp.float32),
    )


def kernel(label, cls_ctx, token_prefix, token_suffix):
    b = label.shape[0]
    num_class = cls_ctx.shape[0]
    info = plsc.get_sparse_core_info()
    nc, ns = info.num_cores, info.num_subcores
    nw = nc * ns
    assert b % nw == 0 and (b // nw) % 8 == 0
    b_per_w = b // nw
    idx = label.astype(jnp.int32)
    ctx = _make_sc_gather(num_class, b, nc, b_per_w)(cls_ctx, idx)
    ctx_t = jnp.transpose(ctx, (1, 0, 2))
    pre_t = jnp.transpose(token_prefix, (1, 0, 2))
    suf_t = jnp.transpose(token_suffix, (1, 0, 2))
    out770 = _make_tc_assemble(b, 128)(ctx_t, pre_t, suf_t)
    return jnp.transpose(out770, (1, 0, 2))


# speedup vs baseline: 1.1264x; 1.1264x over previous
import functools

import jax
import jax.numpy as jnp
from jax import lax
from jax.experimental import pallas as pl
from jax.experimental.pallas import tpu as pltpu
from jax.experimental.pallas import tpu_sc as plsc

N_CLS_CTX = 4
CTX_DIM = 512
CONTEXT_LEN = 77
PREFIX_LEN = 5
SUFFIX_LEN = CONTEXT_LEN - PREFIX_LEN - N_CLS_CTX
CTX_BEG = PREFIX_LEN
SUF_BEG = PREFIX_LEN + N_CLS_CTX


def _make_sc_gather(num_class: int, b: int, nc: int, b_per_w: int):
    @functools.partial(
        pl.kernel,
        mesh=plsc.VectorSubcoreMesh(core_axis_name="c", subcore_axis_name="s"),
        out_type=jax.ShapeDtypeStruct((b, N_CLS_CTX, CTX_DIM), jnp.float32),
        scratch_types=[
            pltpu.VMEM((b_per_w,), jnp.int32),
            pltpu.VMEM((b_per_w, N_CLS_CTX, CTX_DIM), jnp.float32),
            pltpu.SemaphoreType.DMA,
        ],
    )
    def gather(table_hbm, idx_hbm, out_hbm, idx_v, rows_v, sem):
        wid = lax.axis_index("s") * nc + lax.axis_index("c")
        base = wid * b_per_w
        pltpu.sync_copy(idx_hbm.at[pl.ds(base, b_per_w)], idx_v)
        pltpu.async_copy(table_hbm.at[idx_v], rows_v, sem).wait()
        pltpu.sync_copy(rows_v, out_hbm.at[pl.ds(base, b_per_w)])

    return gather


def _assemble_body(ctx_ref, pre_ref, suf_ref, out_ref):
    bb = out_ref.shape[1]
    out_ref[:PREFIX_LEN] = jnp.broadcast_to(
        pre_ref[...], (PREFIX_LEN, bb, CTX_DIM))
    out_ref[CTX_BEG:SUF_BEG] = jnp.transpose(ctx_ref[...], (1, 0, 2))
    out_ref[SUF_BEG:] = jnp.broadcast_to(
        suf_ref[...], (SUFFIX_LEN, bb, CTX_DIM))


def _make_tc_assemble(b: int, bb: int):
    return pl.pallas_call(
        _assemble_body,
        grid=(b // bb,),
        in_specs=[
            pl.BlockSpec((bb, N_CLS_CTX, CTX_DIM), lambda i: (i, 0, 0)),
            pl.BlockSpec((PREFIX_LEN, 1, CTX_DIM), lambda i: (0, 0, 0)),
            pl.BlockSpec((SUFFIX_LEN, 1, CTX_DIM), lambda i: (0, 0, 0)),
        ],
        out_specs=pl.BlockSpec((CONTEXT_LEN, bb, CTX_DIM), lambda i: (0, i, 0)),
        out_shape=jax.ShapeDtypeStruct((CONTEXT_LEN, b, CTX_DIM), jnp.float32),
    )


def kernel(label, cls_ctx, token_prefix, token_suffix):
    b = label.shape[0]
    num_class = cls_ctx.shape[0]
    info = plsc.get_sparse_core_info()
    nc, ns = info.num_cores, info.num_subcores
    nw = nc * ns
    assert b % nw == 0 and (b // nw) % 8 == 0
    b_per_w = b // nw
    idx = label.astype(jnp.int32)
    ctx = _make_sc_gather(num_class, b, nc, b_per_w)(cls_ctx, idx)
    pre_t = jnp.transpose(token_prefix, (1, 0, 2))
    suf_t = jnp.transpose(token_suffix, (1, 0, 2))
    out770 = _make_tc_assemble(b, 64)(ctx, pre_t, suf_t)
    return jnp.transpose(out770, (1, 0, 2))
